# baseline (speedup 1.0000x reference)
import dataclasses
import functools

import jax
import jax.numpy as jnp
from jax import lax
from jax.experimental import pallas as pl
from jax.experimental.pallas import tpu as pltpu
from jax.experimental.pallas import tpu_sc as plsc

NUM_ITEMS = 1000000
D = 32
B = 16384
LANES = 128
BN = 65536
BSUB = 8192
N_BLOCKS = -(-NUM_ITEMS // BN)
SROWS = N_BLOCKS * (BN // LANES)
NC = 2
NS = 16
NW = NC * NS
B_PER_W = B // NW
CHUNK = 128
N_CHUNKS = B_PER_W // CHUNK


def _scores_tc():
    def body(u_ref, t_ref, o_ref):
        u = u_ref[...]
        for rr in range(BN // BSUB):
            sub = t_ref[:, pl.ds(rr * BSUB, BSUB)]
            s = jnp.dot(u, sub, preferred_element_type=jnp.float32)
            sig = 1.0 / (1.0 + jnp.exp(-s))
            rbase = rr * (BSUB // LANES)
            for r in range(BSUB // LANES):
                o_ref[pl.ds(rbase + r, 1), :] = (
                    sig[:, r * LANES:(r + 1) * LANES])

    return pl.pallas_call(
        body,
        grid=(N_BLOCKS,),
        in_specs=[
            pl.BlockSpec((1, D), lambda j: (0, 0)),
            pl.BlockSpec((D, BN), lambda j: (0, j)),
        ],
        out_specs=pl.BlockSpec((BN // LANES, LANES), lambda j: (j, 0)),
        out_shape=jax.ShapeDtypeStruct((SROWS, LANES), jnp.float32),
    )


def _gather_sc():
    mesh = plsc.VectorSubcoreMesh(core_axis_name="c", subcore_axis_name="s")
    cp = pltpu.CompilerParams()
    if "needs_layout_passes" in pltpu.CompilerParams.__dataclass_fields__:
        cp = dataclasses.replace(cp, needs_layout_passes=False)

    @functools.partial(
        pl.kernel,
        mesh=mesh,
        compiler_params=cp,
        out_type=jax.ShapeDtypeStruct((B,), jnp.float32),
        scratch_types=[
            pltpu.VMEM((B_PER_W,), jnp.int32),
            pltpu.VMEM((N_CHUNKS, CHUNK), jnp.int32),
            pltpu.VMEM((B_PER_W,), jnp.int32),
            pltpu.VMEM((B_PER_W, LANES), jnp.float32),
            pltpu.VMEM((B_PER_W,), jnp.float32),
            pltpu.SemaphoreType.DMA,
        ],
    )
    def gather_kernel(idx_hbm, sig_hbm, out_hbm,
                      idx_v, rw_v, ln_v, rows_v, out_v, sem):
        wid = lax.axis_index("s") * NC + lax.axis_index("c")
        base = wid * B_PER_W

        pltpu.sync_copy(idx_hbm.at[wid], idx_v)
        for v in range(B_PER_W // 16):
            iv = idx_v[pl.ds(v * 16, 16)]
            j, o = divmod(v * 16, CHUNK)
            rw_v[j, pl.ds(o, 16)] = iv >> 7
            ln_v[pl.ds(v * 16, 16)] = iv & (LANES - 1)

        copies = []
        for j in range(N_CHUNKS):
            copies.append(pltpu.async_copy(
                sig_hbm.at[rw_v.at[j]],
                rows_v.at[pl.ds(j * CHUNK, CHUNK)],
                sem,
            ))
        for c in copies:
            c.wait()

        lane16 = lax.iota(jnp.int32, 16)
        for g in range(B_PER_W // 16):
            kv = g * 16 + lane16
            cv = ln_v[pl.ds(g * 16, 16)]
            out_v[pl.ds(g * 16, 16)] = plsc.load_gather(rows_v, [kv, cv])

        pltpu.sync_copy(out_v, out_hbm.at[pl.ds(base, B_PER_W)])

    return gather_kernel


_tc_scores = _scores_tc()
_sc_gather = _gather_sc()


def kernel(item_indices, item_table, user_embedding):
    idx = item_indices.astype(jnp.int32).reshape(NW, B_PER_W)
    sig = _tc_scores(user_embedding.reshape(1, D), item_table.T)
    return _sc_gather(idx, sig)


# speedup vs baseline: 1.0036x; 1.0036x over previous
import dataclasses
import functools

import jax
import jax.numpy as jnp
from jax import lax
from jax.experimental import pallas as pl
from jax.experimental.pallas import tpu as pltpu
from jax.experimental.pallas import tpu_sc as plsc

NUM_ITEMS = 1000000
D = 32
B = 16384
LANES = 128
BN = 65536
BSUB = 16384
N_BLOCKS = -(-NUM_ITEMS // BN)
SROWS = N_BLOCKS * (BN // LANES)
NC = 2
NS = 16
NW = NC * NS
B_PER_W = B // NW
CHUNK = 128
N_CHUNKS = B_PER_W // CHUNK


def _scores_tc():
    def body(u_ref, t_ref, o_ref):
        u = u_ref[...]
        for rr in range(BN // BSUB):
            sub = t_ref[:, pl.ds(rr * BSUB, BSUB)]
            s = jnp.dot(u, sub, preferred_element_type=jnp.float32)
            sig = 1.0 / (1.0 + jnp.exp(-s))
            rbase = rr * (BSUB // LANES)
            for r in range(BSUB // LANES):
                o_ref[pl.ds(rbase + r, 1), :] = (
                    sig[:, r * LANES:(r + 1) * LANES])

    return pl.pallas_call(
        body,
        grid=(N_BLOCKS,),
        in_specs=[
            pl.BlockSpec((1, D), lambda j: (0, 0)),
            pl.BlockSpec((D, BN), lambda j: (0, j)),
        ],
        out_specs=pl.BlockSpec((BN // LANES, LANES), lambda j: (j, 0)),
        out_shape=jax.ShapeDtypeStruct((SROWS, LANES), jnp.float32),
    )


def _gather_sc():
    mesh = plsc.VectorSubcoreMesh(core_axis_name="c", subcore_axis_name="s")
    cp = pltpu.CompilerParams()
    if "needs_layout_passes" in pltpu.CompilerParams.__dataclass_fields__:
        cp = dataclasses.replace(cp, needs_layout_passes=False)

    @functools.partial(
        pl.kernel,
        mesh=mesh,
        compiler_params=cp,
        out_type=jax.ShapeDtypeStruct((B,), jnp.float32),
        scratch_types=[
            pltpu.VMEM((B_PER_W,), jnp.int32),
            pltpu.VMEM((N_CHUNKS, CHUNK), jnp.int32),
            pltpu.VMEM((B_PER_W,), jnp.int32),
            pltpu.VMEM((B_PER_W, LANES), jnp.float32),
            pltpu.VMEM((B_PER_W,), jnp.float32),
            pltpu.SemaphoreType.DMA,
        ],
    )
    def gather_kernel(idx_hbm, sig_hbm, out_hbm,
                      idx_v, rw_v, ln_v, rows_v, out_v, sem):
        wid = lax.axis_index("s") * NC + lax.axis_index("c")
        base = wid * B_PER_W

        pltpu.sync_copy(idx_hbm.at[wid], idx_v)
        for v in range(B_PER_W // 16):
            iv = idx_v[pl.ds(v * 16, 16)]
            j, o = divmod(v * 16, CHUNK)
            rw_v[j, pl.ds(o, 16)] = iv >> 7
            ln_v[pl.ds(v * 16, 16)] = iv & (LANES - 1)

        copies = []
        for j in range(N_CHUNKS):
            copies.append(pltpu.async_copy(
                sig_hbm.at[rw_v.at[j]],
                rows_v.at[pl.ds(j * CHUNK, CHUNK)],
                sem,
            ))
        for c in copies:
            c.wait()

        lane16 = lax.iota(jnp.int32, 16)
        for g in range(B_PER_W // 16):
            kv = g * 16 + lane16
            cv = ln_v[pl.ds(g * 16, 16)]
            out_v[pl.ds(g * 16, 16)] = plsc.load_gather(rows_v, [kv, cv])

        pltpu.sync_copy(out_v, out_hbm.at[pl.ds(base, B_PER_W)])

    return gather_kernel


_tc_scores = _scores_tc()
_sc_gather = _gather_sc()


def kernel(item_indices, item_table, user_embedding):
    idx = item_indices.astype(jnp.int32).reshape(NW, B_PER_W)
    sig = _tc_scores(user_embedding.reshape(1, D), item_table.T)
    return _sc_gather(idx, sig)
